# Initial kernel scaffold; baseline (speedup 1.0000x reference)
import jax
import jax.numpy as jnp
from jax.experimental import pallas as pl


def kernel(x, uu, vv, mask):
    raise NotImplementedError("write your pallas kernel here")


# speedup vs baseline: 1.7682x; 1.7682x over previous
import functools
import math

import jax
import jax.numpy as jnp
from jax.experimental import pallas as pl
from jax.experimental.pallas import tpu as pltpu


def _tap_conv_kernel(x_ref, w_ref, o_ref, *, K, H, W):
    HW = H * W
    x = x_ref[0]
    pos = jax.lax.broadcasted_iota(jnp.int32, (1, HW), 1)
    r = pos // W
    c = pos - r * W
    half = K // 2
    acc = None
    for kh in range(K):
        for kw in range(K):
            t = kh * K + kw
            off = (kh - half) * W + (kw - half)
            rolled = x if off == 0 else pltpu.roll(x, shift=(-off) % HW, axis=1)
            conds = []
            if kh - half < 0:
                conds.append(r >= half - kh)
            if kh - half > 0:
                conds.append(r < H - (kh - half))
            if kw - half < 0:
                conds.append(c >= half - kw)
            if kw - half > 0:
                conds.append(c < W - (kw - half))
            if conds:
                v = conds[0]
                for extra in conds[1:]:
                    v = jnp.logical_and(v, extra)
                rolled = jnp.where(v, rolled, 0.0)
            xt = rolled.astype(jnp.bfloat16)
            p = jnp.dot(w_ref[t], xt, preferred_element_type=jnp.float32)
            acc = p if acc is None else acc + p
    o_ref[0] = acc


def kernel(x, uu, vv, mask):
    B, d_i, H, W = x.shape
    KK = uu.shape[0]
    K = math.isqrt(KK)
    d_o = vv.shape[1] // d_i
    HW = H * W

    w_oikk = (uu @ vv + mask).reshape(d_o, d_i, K, K)
    w = jnp.transpose(w_oikk, (2, 3, 0, 1)).reshape(KK, d_o, d_i)
    w = w.astype(jnp.bfloat16)

    xf = x.reshape(B, d_i, HW)
    out = pl.pallas_call(
        functools.partial(_tap_conv_kernel, K=K, H=H, W=W),
        out_shape=jax.ShapeDtypeStruct((B, d_o, HW), jnp.float32),
        grid=(B,),
        in_specs=[
            pl.BlockSpec((1, d_i, HW), lambda i: (i, 0, 0)),
            pl.BlockSpec((KK, d_o, d_i), lambda i: (0, 0, 0)),
        ],
        out_specs=pl.BlockSpec((1, d_o, HW), lambda i: (i, 0, 0)),
        compiler_params=pltpu.CompilerParams(dimension_semantics=("parallel",)),
    )(xf, w)
    return out.reshape(B, d_o, H, W).astype(x.dtype)


# speedup vs baseline: 6.9300x; 3.9192x over previous
import functools
import math

import jax
import jax.numpy as jnp
from jax.experimental import pallas as pl
from jax.experimental.pallas import tpu as pltpu


def _conv_kernel(x_ref, f2_ref, o_ref, wt_ref, *, K, H, W, d_i, d_o):
    KK = K * K
    HW = H * W
    half = K // 2

    @pl.when(pl.program_id(1) == 0)
    def _build_weights():
        f2 = f2_ref[...]
        k_iota = jax.lax.broadcasted_iota(jnp.int32, (d_i * KK, d_i), 0)
        i_iota = jax.lax.broadcasted_iota(jnp.int32, (d_i * KK, d_i), 1)
        base = i_iota * KK
        for t in range(KK):
            sel = (k_iota == base + t).astype(jnp.bfloat16)
            wt = jnp.dot(f2, sel, preferred_element_type=jnp.float32)
            wt_ref[t] = wt.astype(jnp.bfloat16)

    x = x_ref[0]
    pos = jax.lax.broadcasted_iota(jnp.int32, (1, HW), 1)
    r = pos // W
    c = pos - r * W
    acc = None
    for kh in range(K):
        for kw in range(K):
            t = kh * K + kw
            off = (kh - half) * W + (kw - half)
            rolled = x if off == 0 else pltpu.roll(x, shift=(-off) % HW, axis=1)
            conds = []
            if kh - half < 0:
                conds.append(r >= half - kh)
            if kh - half > 0:
                conds.append(r < H - (kh - half))
            if kw - half < 0:
                conds.append(c >= half - kw)
            if kw - half > 0:
                conds.append(c < W - (kw - half))
            if conds:
                v = conds[0]
                for extra in conds[1:]:
                    v = jnp.logical_and(v, extra)
                rolled = jnp.where(v, rolled, 0.0)
            xt = rolled.astype(jnp.bfloat16)
            p = jnp.dot(wt_ref[t], xt, preferred_element_type=jnp.float32)
            acc = p if acc is None else acc + p
    o_ref[0] = acc


def kernel(x, uu, vv, mask):
    B, d_i, H, W = x.shape
    KK = uu.shape[0]
    K = math.isqrt(KK)
    d_o = vv.shape[1] // d_i
    HW = H * W

    f2 = (uu @ vv + mask).astype(jnp.bfloat16).reshape(d_o, d_i * KK)

    xf = x.reshape(B, d_i, HW)
    n_cores = 2 if B % 2 == 0 else 1
    per = B // n_cores
    out = pl.pallas_call(
        functools.partial(_conv_kernel, K=K, H=H, W=W, d_i=d_i, d_o=d_o),
        out_shape=jax.ShapeDtypeStruct((B, d_o, HW), jnp.float32),
        grid=(n_cores, per),
        in_specs=[
            pl.BlockSpec((1, d_i, HW), lambda cc, j: (cc * per + j, 0, 0)),
            pl.BlockSpec((d_o, d_i * KK), lambda cc, j: (0, 0)),
        ],
        out_specs=pl.BlockSpec((1, d_o, HW), lambda cc, j: (cc * per + j, 0, 0)),
        scratch_shapes=[pltpu.VMEM((KK, d_o, d_i), jnp.bfloat16)],
        compiler_params=pltpu.CompilerParams(
            dimension_semantics=("parallel", "arbitrary")),
    )(xf, f2)
    return out.reshape(B, d_o, H, W).astype(x.dtype)
